# Initial kernel scaffold; baseline (speedup 1.0000x reference)
import jax
import jax.numpy as jnp
from jax.experimental import pallas as pl


def kernel(ids, masks, DF_table):
    raise NotImplementedError("write your pallas kernel here")


# speedup vs baseline: 56.7331x; 56.7331x over previous
import functools

import jax
import jax.numpy as jnp
from jax import lax
from jax.experimental import pallas as pl
from jax.experimental.pallas import tpu as pltpu
from jax.experimental.pallas import tpu_sc as plsc

K1 = 1.2
K3 = 8.0
B_PARAM = 0.75
N_DOCS = 8800000.0
LAVE = 250.0
VOCAB = 100000
BATCH = 1024
L_SEQ = 200

NC = 2
NS = 16
NW = NC * NS
LANES = 16
COLS = 7
HIST_PAD = 100352
VREGS_B = BATCH // LANES
UNROLL = 8

_DENOM = K1 * (1.0 - B_PARAM + B_PARAM * L_SEQ / LAVE)

_mesh = plsc.VectorSubcoreMesh(core_axis_name="c", subcore_axis_name="s")
_sc_params = pltpu.CompilerParams(needs_layout_passes=False)


def _idf_body(df_ref, out_ref):
    df = df_ref[...]
    out_ref[...] = jnp.log2((N_DOCS - df + 0.5) / (df + 0.5))


def _idf_table(df2d):
    return pl.pallas_call(
        _idf_body,
        out_shape=jax.ShapeDtypeStruct(df2d.shape, jnp.float32),
    )(df2d)


@functools.partial(
    pl.kernel,
    mesh=_mesh,
    out_type=(
        jax.ShapeDtypeStruct((NW, BATCH), jnp.int32),
        jax.ShapeDtypeStruct((NW, BATCH), jnp.int32),
    ),
    scratch_types=[
        pltpu.VMEM((HIST_PAD,), jnp.int32),
        pltpu.VMEM((BATCH,), jnp.int32),
        pltpu.VMEM((BATCH,), jnp.int32),
        pltpu.VMEM((BATCH,), jnp.int32),
        pltpu.VMEM((BATCH,), jnp.int32),
    ],
    compiler_params=_sc_params,
)
def _count_kernel(qT, pT, outq, outp, hist, qcol, pcol, accq, accp):
    wid = lax.axis_index("s") * NC + lax.axis_index("c")
    ones = jnp.full((LANES,), 1, jnp.int32)
    neg_ones = jnp.full((LANES,), -1, jnp.int32)
    zeros16 = jnp.zeros((LANES,), jnp.int32)

    def zero_hist(i, carry):
        for u in range(16):
            hist[pl.ds(i * 256 + u * 16, LANES)] = zeros16
        return carry

    lax.fori_loop(0, HIST_PAD // 256, zero_hist, 0)

    def zero_acc(i, carry):
        accq[pl.ds(i * LANES, LANES)] = zeros16
        accp[pl.ds(i * LANES, LANES)] = zeros16
        return carry

    lax.fori_loop(0, VREGS_B, zero_acc, 0)

    def col_body(k, carry):
        j = wid * COLS + k

        @pl.when(j < L_SEQ)
        def _():
            pltpu.sync_copy(qT.at[j], qcol)
            pltpu.sync_copy(pT.at[j], pcol)

            def insert(col, delta):
                def body(i, c):
                    for u in range(UNROLL):
                        idx = col[pl.ds((i * UNROLL + u) * LANES, LANES)]
                        plsc.addupdate_scatter(hist, [idx], delta)
                    return c

                lax.fori_loop(0, VREGS_B // UNROLL, body, 0)

            def gather_into(acc):
                def body(i, c):
                    for u in range(UNROLL):
                        sl = pl.ds((i * UNROLL + u) * LANES, LANES)
                        idx = qcol[sl]
                        acc[sl] = acc[sl] + plsc.load_gather(hist, [idx])
                    return c

                lax.fori_loop(0, VREGS_B // UNROLL, body, 0)

            insert(qcol, ones)
            gather_into(accq)
            insert(qcol, neg_ones)
            insert(pcol, ones)
            gather_into(accp)
            insert(pcol, neg_ones)

        return carry

    lax.fori_loop(0, COLS, col_body, 0)

    pltpu.sync_copy(accq, outq.at[wid])
    pltpu.sync_copy(accp, outp.at[wid])


def _w_body(q_ref, p_ref, out_ref):
    qtf = jnp.sum(q_ref[...], axis=0).astype(jnp.float32)
    ptf = jnp.sum(p_ref[...], axis=0).astype(jnp.float32)
    out_ref[...] = (qtf / (K3 + qtf)) * (K1 * ptf / (ptf + _DENOM))


def _weights(q_parts, p_parts):
    return pl.pallas_call(
        _w_body,
        out_shape=jax.ShapeDtypeStruct((8, 128), jnp.float32),
    )(q_parts, p_parts)


@functools.partial(
    pl.kernel,
    mesh=_mesh,
    out_type=jax.ShapeDtypeStruct((NW, LANES), jnp.float32),
    scratch_types=[
        pltpu.VMEM((HIST_PAD,), jnp.float32),
        pltpu.VMEM((BATCH,), jnp.float32),
        pltpu.VMEM((BATCH,), jnp.int32),
        pltpu.VMEM((LANES,), jnp.float32),
    ],
    compiler_params=_sc_params,
)
def _score_kernel(qT, w_hbm, idf_hbm, out, idf_v, wv, qcol, outv):
    wid = lax.axis_index("s") * NC + lax.axis_index("c")
    lanes = jnp.arange(LANES, dtype=jnp.int32)

    pltpu.sync_copy(idf_hbm, idf_v)
    pltpu.sync_copy(w_hbm, wv)
    outv[...] = jnp.zeros((LANES,), jnp.float32)

    def col_body(k, carry):
        j = wid * COLS + k

        @pl.when(j < L_SEQ)
        def _():
            pltpu.sync_copy(qT.at[j], qcol)

            def body(i, acc):
                for u in range(UNROLL):
                    sl = pl.ds((i * UNROLL + u) * LANES, LANES)
                    idx = qcol[sl]
                    acc = acc + wv[sl] * plsc.load_gather(idf_v, [idx])
                return acc

            acc = lax.fori_loop(
                0, VREGS_B // UNROLL, body, jnp.zeros((LANES,), jnp.float32)
            )
            sc = jnp.sum(acc)
            outv[...] = jnp.where(lanes == k, sc, outv[...])

        return carry

    lax.fori_loop(0, COLS, col_body, 0)

    outv[...] = 1.0 / (1.0 + jnp.exp(-outv[...]))
    pltpu.sync_copy(outv, out.at[wid])


@jax.jit
def kernel(ids, masks, DF_table):
    del masks
    qT = jnp.transpose(ids[0])
    pT = jnp.transpose(ids[1])

    df_pad = jnp.pad(DF_table, (0, HIST_PAD - VOCAB), constant_values=1.0)
    idf_flat = _idf_table(df_pad.reshape(HIST_PAD // 128, 128)).reshape(HIST_PAD)

    q_parts, p_parts = _count_kernel(qT, pT)
    w = _weights(
        q_parts.reshape(NW, 8, 128), p_parts.reshape(NW, 8, 128)
    ).reshape(BATCH)

    out32 = _score_kernel(qT, w, idf_flat)
    return out32[:, :COLS].reshape(-1)[:L_SEQ]


# speedup vs baseline: 81.9395x; 1.4443x over previous
import functools

import jax
import jax.numpy as jnp
from jax import lax
from jax.experimental import pallas as pl
from jax.experimental.pallas import tpu as pltpu
from jax.experimental.pallas import tpu_sc as plsc

K1 = 1.2
K3 = 8.0
B_PARAM = 0.75
N_DOCS = 8800000.0
LAVE = 250.0
VOCAB = 100000
BATCH = 1024
L_SEQ = 200

NC = 2
NS = 16
NW = NC * NS
LANES = 16
COLS = 7
LPAD = NW * COLS
HIST_PAD = 100352
VREGS_B = BATCH // LANES
UNROLL = 16
IDX_CHUNK = 128

_DENOM = K1 * (1.0 - B_PARAM + B_PARAM * L_SEQ / LAVE)

_mesh = plsc.VectorSubcoreMesh(core_axis_name="c", subcore_axis_name="s")
_sc_params = pltpu.CompilerParams(needs_layout_passes=False)


@functools.partial(
    pl.kernel,
    mesh=_mesh,
    out_type=(
        jax.ShapeDtypeStruct((NW, BATCH), jnp.int32),
        jax.ShapeDtypeStruct((NW, BATCH), jnp.int32),
        jax.ShapeDtypeStruct((LPAD, BATCH), jnp.float32),
    ),
    scratch_types=[
        pltpu.VMEM((HIST_PAD,), jnp.int32),
        pltpu.VMEM((BATCH,), jnp.int32),
        pltpu.VMEM((BATCH,), jnp.int32),
        pltpu.VMEM((BATCH,), jnp.int32),
        pltpu.VMEM((BATCH,), jnp.int32),
        pltpu.VMEM((BATCH,), jnp.float32),
        pltpu.SemaphoreType.DMA,
    ],
    compiler_params=_sc_params,
)
def _count_kernel(qT, pT, df_hbm, outq, outp, dfq_hbm,
                  hist, qcol, pcol, accq, accp, dfq, sem):
    wid = lax.axis_index("s") * NC + lax.axis_index("c")
    ones = jnp.full((LANES,), 1, jnp.int32)
    neg_ones = jnp.full((LANES,), -1, jnp.int32)
    zeros16 = jnp.zeros((LANES,), jnp.int32)

    def zero_hist(i, carry):
        for u in range(16):
            hist[pl.ds(i * 256 + u * 16, LANES)] = zeros16
        return carry

    lax.fori_loop(0, HIST_PAD // 256, zero_hist, 0)

    def zero_acc(i, carry):
        accq[pl.ds(i * LANES, LANES)] = zeros16
        accp[pl.ds(i * LANES, LANES)] = zeros16
        dfq[pl.ds(i * LANES, LANES)] = jnp.zeros((LANES,), jnp.float32)
        return carry

    lax.fori_loop(0, VREGS_B, zero_acc, 0)

    def col_body(k, carry):
        j = wid * COLS + k

        @pl.when(j < L_SEQ)
        def _():
            pltpu.sync_copy(qT.at[j], qcol)
            pltpu.sync_copy(pT.at[j], pcol)

            copies = []
            for c in range(BATCH // IDX_CHUNK):
                sl = pl.ds(c * IDX_CHUNK, IDX_CHUNK)
                copies.append(
                    pltpu.async_copy(df_hbm.at[qcol.at[sl]], dfq.at[sl], sem)
                )

            def insert(col, delta):
                def body(i, c):
                    for u in range(UNROLL):
                        idx = col[pl.ds((i * UNROLL + u) * LANES, LANES)]
                        plsc.addupdate_scatter(hist, [idx], delta)
                    return c

                lax.fori_loop(0, VREGS_B // UNROLL, body, 0)

            def gather_into(acc):
                def body(i, c):
                    for u in range(UNROLL):
                        sl = pl.ds((i * UNROLL + u) * LANES, LANES)
                        idx = qcol[sl]
                        acc[sl] = acc[sl] + plsc.load_gather(hist, [idx])
                    return c

                lax.fori_loop(0, VREGS_B // UNROLL, body, 0)

            insert(qcol, ones)
            gather_into(accq)
            insert(qcol, neg_ones)
            insert(pcol, ones)
            gather_into(accp)
            insert(pcol, neg_ones)

            for cp_ in copies:
                cp_.wait()

        pltpu.sync_copy(dfq, dfq_hbm.at[j])
        return carry

    lax.fori_loop(0, COLS, col_body, 0)

    pltpu.sync_copy(accq, outq.at[wid])
    pltpu.sync_copy(accp, outp.at[wid])


def _finish_body(q_ref, p_ref, dfq_ref, out_ref):
    qtf = jnp.sum(q_ref[...], axis=0).astype(jnp.float32)
    ptf = jnp.sum(p_ref[...], axis=0).astype(jnp.float32)
    w = (qtf / (K3 + qtf)) * (K1 * ptf / (ptf + _DENOM))
    df = dfq_ref[...]
    idf = jnp.log2((N_DOCS - df + 0.5) / (df + 0.5))
    score = jnp.sum(idf * w[None, :], axis=1)
    out_ref[...] = (1.0 / (1.0 + jnp.exp(-score)))[None, :]


def _finish(q_parts, p_parts, dfq):
    return pl.pallas_call(
        _finish_body,
        out_shape=jax.ShapeDtypeStruct((1, LPAD), jnp.float32),
    )(q_parts, p_parts, dfq)


@jax.jit
def kernel(ids, masks, DF_table):
    del masks
    qT = jnp.transpose(ids[0])
    pT = jnp.transpose(ids[1])
    q_parts, p_parts, dfq = _count_kernel(qT, pT, DF_table)
    return _finish(q_parts, p_parts, dfq)[0, :L_SEQ]


# speedup vs baseline: 85.2402x; 1.0403x over previous
import functools

import jax
import jax.numpy as jnp
from jax import lax
from jax.experimental import pallas as pl
from jax.experimental.pallas import tpu as pltpu
from jax.experimental.pallas import tpu_sc as plsc

K1 = 1.2
K3 = 8.0
B_PARAM = 0.75
N_DOCS = 8800000.0
LAVE = 250.0
VOCAB = 100000
BATCH = 1024
L_SEQ = 200

NC = 2
NS = 16
NW = NC * NS
LANES = 16
COLS = 7
LPAD = NW * COLS
HIST_PAD = 100352
VREGS_B = BATCH // LANES
UNROLL = 16
IDX_CHUNK = 128
PHI = 65536

_DENOM = K1 * (1.0 - B_PARAM + B_PARAM * L_SEQ / LAVE)

_mesh = plsc.VectorSubcoreMesh(core_axis_name="c", subcore_axis_name="s")
_sc_params = pltpu.CompilerParams(needs_layout_passes=False)


@functools.partial(
    pl.kernel,
    mesh=_mesh,
    out_type=(
        jax.ShapeDtypeStruct((NW, BATCH), jnp.int32),
        jax.ShapeDtypeStruct((NW, BATCH), jnp.int32),
        jax.ShapeDtypeStruct((LPAD, BATCH), jnp.float32),
    ),
    scratch_types=[
        pltpu.VMEM((HIST_PAD,), jnp.int32),
        pltpu.VMEM((BATCH,), jnp.int32),
        pltpu.VMEM((BATCH,), jnp.int32),
        pltpu.VMEM((BATCH,), jnp.int32),
        pltpu.VMEM((BATCH,), jnp.int32),
        pltpu.VMEM((BATCH,), jnp.int32),
        pltpu.VMEM((BATCH,), jnp.int32),
        pltpu.VMEM((BATCH,), jnp.float32),
        pltpu.VMEM((BATCH,), jnp.float32),
        pltpu.SemaphoreType.DMA,
        pltpu.SemaphoreType.DMA,
        pltpu.SemaphoreType.DMA,
    ],
    compiler_params=_sc_params,
)
def _count_kernel(qT, pT, df_hbm, outq, outp, dfq_hbm,
                  hist, qcol_a, qcol_b, pcol_a, pcol_b, accq, accp,
                  dfq_a, dfq_b, sem_in, sem_g, sem_out):
    qcols = (qcol_a, qcol_b)
    pcols = (pcol_a, pcol_b)
    dfqs = (dfq_a, dfq_b)
    wid = lax.axis_index("s") * NC + lax.axis_index("c")
    ones = jnp.full((LANES,), 1, jnp.int32)
    neg_ones = jnp.full((LANES,), -1, jnp.int32)
    p_ones = jnp.full((LANES,), PHI, jnp.int32)
    neg_p_ones = jnp.full((LANES,), -PHI, jnp.int32)
    zeros16 = jnp.zeros((LANES,), jnp.int32)

    def fire_loads(k):
        j = wid * COLS + k

        @pl.when(j < L_SEQ)
        def _():
            pltpu.async_copy(qT.at[j], qcols[k % 2], sem_in)
            pltpu.async_copy(pT.at[j], pcols[k % 2], sem_in)

    def wait_loads(k):
        j = wid * COLS + k

        @pl.when(j < L_SEQ)
        def _():
            pltpu.make_async_copy(qT.at[j], qcols[k % 2], sem_in).wait()
            pltpu.make_async_copy(pT.at[j], pcols[k % 2], sem_in).wait()

    fire_loads(0)

    def zero_hist(i, carry):
        for u in range(16):
            hist[pl.ds(i * 256 + u * 16, LANES)] = zeros16
        return carry

    lax.fori_loop(0, HIST_PAD // 256, zero_hist, 0)

    def zero_acc(i, carry):
        accq[pl.ds(i * LANES, LANES)] = zeros16
        accp[pl.ds(i * LANES, LANES)] = zeros16
        dfq_a[pl.ds(i * LANES, LANES)] = jnp.zeros((LANES,), jnp.float32)
        dfq_b[pl.ds(i * LANES, LANES)] = jnp.zeros((LANES,), jnp.float32)
        return carry

    lax.fori_loop(0, VREGS_B, zero_acc, 0)

    for k in range(COLS):
        j = wid * COLS + k
        qcol = qcols[k % 2]
        pcol = pcols[k % 2]
        dfq = dfqs[k % 2]

        wait_loads(k)
        if k + 1 < COLS:
            fire_loads(k + 1)
        if k >= 2:
            jm2 = wid * COLS + (k - 2)
            pltpu.make_async_copy(dfq, dfq_hbm.at[jm2], sem_out).wait()

        @pl.when(j < L_SEQ)
        def _():
            for c in range(BATCH // IDX_CHUNK):
                sl = pl.ds(c * IDX_CHUNK, IDX_CHUNK)
                pltpu.async_copy(df_hbm.at[qcol.at[sl]], dfq.at[sl], sem_g)

            def scatter_q(delta):
                def body(i, c):
                    for u in range(UNROLL):
                        idx = qcol[pl.ds(i * (UNROLL * LANES) + u * LANES,
                                         LANES)]
                        plsc.addupdate_scatter(hist, [idx], delta)
                    return c

                lax.fori_loop(0, VREGS_B // UNROLL, body, 0)

            def scatter_p(delta):
                def body(i, c):
                    for u in range(UNROLL):
                        idx = pcol[pl.ds(i * (UNROLL * LANES) + u * LANES,
                                         LANES)]
                        plsc.addupdate_scatter(hist, [idx], delta)
                    return c

                lax.fori_loop(0, VREGS_B // UNROLL, body, 0)

            def gather_both(i, c):
                for u in range(UNROLL):
                    sl = pl.ds(i * (UNROLL * LANES) + u * LANES, LANES)
                    idx = qcol[sl]
                    g = plsc.load_gather(hist, [idx])
                    accq[sl] = accq[sl] + (g & (PHI - 1))
                    accp[sl] = accp[sl] + (g >> 16)
                return c

            scatter_q(ones)
            scatter_p(p_ones)
            lax.fori_loop(0, VREGS_B // UNROLL, gather_both, 0)
            if k + 1 < COLS:
                scatter_q(neg_ones)
                scatter_p(neg_p_ones)

            for c in range(BATCH // IDX_CHUNK):
                sl = pl.ds(c * IDX_CHUNK, IDX_CHUNK)
                pltpu.make_async_copy(
                    df_hbm.at[qcol.at[sl]], dfq.at[sl], sem_g
                ).wait()

        pltpu.async_copy(dfq, dfq_hbm.at[j], sem_out)

    for k in (COLS - 2, COLS - 1):
        j = wid * COLS + k
        pltpu.make_async_copy(dfqs[k % 2], dfq_hbm.at[j], sem_out).wait()

    pltpu.sync_copy(accq, outq.at[wid])
    pltpu.sync_copy(accp, outp.at[wid])


def _finish_body(q_ref, p_ref, dfq_ref, out_ref):
    qtf = jnp.sum(q_ref[...], axis=0).astype(jnp.float32)
    ptf = jnp.sum(p_ref[...], axis=0).astype(jnp.float32)
    w = (qtf / (K3 + qtf)) * (K1 * ptf / (ptf + _DENOM))
    df = dfq_ref[...]
    idf = jnp.log2((N_DOCS - df + 0.5) / (df + 0.5))
    score = jnp.sum(idf * w[None, :], axis=1)
    out_ref[...] = (1.0 / (1.0 + jnp.exp(-score)))[None, :]


def _finish(q_parts, p_parts, dfq):
    return pl.pallas_call(
        _finish_body,
        out_shape=jax.ShapeDtypeStruct((1, LPAD), jnp.float32),
    )(q_parts, p_parts, dfq)


@jax.jit
def kernel(ids, masks, DF_table):
    del masks
    qT = jnp.transpose(ids[0])
    pT = jnp.transpose(ids[1])
    q_parts, p_parts, dfq = _count_kernel(qT, pT, DF_table)
    return _finish(q_parts, p_parts, dfq)[0, :L_SEQ]


# speedup vs baseline: 85.3188x; 1.0009x over previous
import functools

import jax
import jax.numpy as jnp
from jax import lax
from jax.experimental import pallas as pl
from jax.experimental.pallas import tpu as pltpu
from jax.experimental.pallas import tpu_sc as plsc

K1 = 1.2
K3 = 8.0
B_PARAM = 0.75
N_DOCS = 8800000.0
LAVE = 250.0
VOCAB = 100000
BATCH = 1024
L_SEQ = 200

NC = 2
NS = 16
NW = NC * NS
LANES = 16
COLS = 7
LPAD = NW * COLS
HIST_PAD = 100352
VREGS_B = BATCH // LANES
UNROLL = 16
IDX_CHUNK = 128
PHI = 65536

_DENOM = K1 * (1.0 - B_PARAM + B_PARAM * L_SEQ / LAVE)

_mesh = plsc.VectorSubcoreMesh(core_axis_name="c", subcore_axis_name="s")
_sc_params = pltpu.CompilerParams(
    needs_layout_passes=False,
    disable_bounds_checks=True,
    skip_device_barrier=True,
)


@functools.partial(
    pl.kernel,
    mesh=_mesh,
    out_type=(
        jax.ShapeDtypeStruct((NW, BATCH), jnp.int32),
        jax.ShapeDtypeStruct((NW, BATCH), jnp.int32),
        jax.ShapeDtypeStruct((LPAD, BATCH), jnp.float32),
    ),
    scratch_types=[
        pltpu.VMEM((HIST_PAD,), jnp.int32),
        pltpu.VMEM((BATCH,), jnp.int32),
        pltpu.VMEM((BATCH,), jnp.int32),
        pltpu.VMEM((BATCH,), jnp.int32),
        pltpu.VMEM((BATCH,), jnp.int32),
        pltpu.VMEM((BATCH,), jnp.int32),
        pltpu.VMEM((BATCH,), jnp.int32),
        pltpu.VMEM((BATCH,), jnp.float32),
        pltpu.VMEM((BATCH,), jnp.float32),
        pltpu.SemaphoreType.DMA,
        pltpu.SemaphoreType.DMA,
        pltpu.SemaphoreType.DMA,
    ],
    compiler_params=_sc_params,
)
def _count_kernel(qT, pT, df_hbm, outq, outp, dfq_hbm,
                  hist, qcol_a, qcol_b, pcol_a, pcol_b, accq, accp,
                  dfq_a, dfq_b, sem_in, sem_g, sem_out):
    qcols = (qcol_a, qcol_b)
    pcols = (pcol_a, pcol_b)
    dfqs = (dfq_a, dfq_b)
    wid = lax.axis_index("s") * NC + lax.axis_index("c")
    ones = jnp.full((LANES,), 1, jnp.int32)
    neg_ones = jnp.full((LANES,), -1, jnp.int32)
    p_ones = jnp.full((LANES,), PHI, jnp.int32)
    neg_p_ones = jnp.full((LANES,), -PHI, jnp.int32)
    zeros16 = jnp.zeros((LANES,), jnp.int32)

    def fire_loads(k):
        j = wid * COLS + k

        @pl.when(j < L_SEQ)
        def _():
            pltpu.async_copy(qT.at[j], qcols[k % 2], sem_in)
            pltpu.async_copy(pT.at[j], pcols[k % 2], sem_in)

    def wait_loads(k):
        j = wid * COLS + k

        @pl.when(j < L_SEQ)
        def _():
            pltpu.make_async_copy(qT.at[j], qcols[k % 2], sem_in).wait()
            pltpu.make_async_copy(pT.at[j], pcols[k % 2], sem_in).wait()

    fire_loads(0)

    def zero_hist(i, carry):
        for u in range(16):
            hist[pl.ds(i * 256 + u * 16, LANES)] = zeros16
        return carry

    lax.fori_loop(0, HIST_PAD // 256, zero_hist, 0)

    def zero_acc(i, carry):
        accq[pl.ds(i * LANES, LANES)] = zeros16
        accp[pl.ds(i * LANES, LANES)] = zeros16
        dfq_a[pl.ds(i * LANES, LANES)] = jnp.zeros((LANES,), jnp.float32)
        dfq_b[pl.ds(i * LANES, LANES)] = jnp.zeros((LANES,), jnp.float32)
        return carry

    lax.fori_loop(0, VREGS_B, zero_acc, 0)

    for k in range(COLS):
        j = wid * COLS + k
        qcol = qcols[k % 2]
        pcol = pcols[k % 2]
        dfq = dfqs[k % 2]

        wait_loads(k)
        if k + 1 < COLS:
            fire_loads(k + 1)
        if k >= 2:
            jm2 = wid * COLS + (k - 2)
            pltpu.make_async_copy(dfq, dfq_hbm.at[jm2], sem_out).wait()

        @pl.when(j < L_SEQ)
        def _():
            for c in range(BATCH // IDX_CHUNK):
                sl = pl.ds(c * IDX_CHUNK, IDX_CHUNK)
                pltpu.async_copy(df_hbm.at[qcol.at[sl]], dfq.at[sl], sem_g)

            def scatter_q(delta):
                def body(i, c):
                    for u in range(UNROLL):
                        idx = qcol[pl.ds(i * (UNROLL * LANES) + u * LANES,
                                         LANES)]
                        plsc.addupdate_scatter(hist, [idx], delta)
                    return c

                lax.fori_loop(0, VREGS_B // UNROLL, body, 0)

            def scatter_p(delta):
                def body(i, c):
                    for u in range(UNROLL):
                        idx = pcol[pl.ds(i * (UNROLL * LANES) + u * LANES,
                                         LANES)]
                        plsc.addupdate_scatter(hist, [idx], delta)
                    return c

                lax.fori_loop(0, VREGS_B // UNROLL, body, 0)

            def gather_both(i, c):
                for u in range(UNROLL):
                    sl = pl.ds(i * (UNROLL * LANES) + u * LANES, LANES)
                    idx = qcol[sl]
                    g = plsc.load_gather(hist, [idx])
                    accq[sl] = accq[sl] + (g & (PHI - 1))
                    accp[sl] = accp[sl] + (g >> 16)
                return c

            scatter_q(ones)
            scatter_p(p_ones)
            lax.fori_loop(0, VREGS_B // UNROLL, gather_both, 0)
            if k + 1 < COLS:
                scatter_q(neg_ones)
                scatter_p(neg_p_ones)

            for c in range(BATCH // IDX_CHUNK):
                sl = pl.ds(c * IDX_CHUNK, IDX_CHUNK)
                pltpu.make_async_copy(
                    df_hbm.at[qcol.at[sl]], dfq.at[sl], sem_g
                ).wait()

        pltpu.async_copy(dfq, dfq_hbm.at[j], sem_out)

    for k in (COLS - 2, COLS - 1):
        j = wid * COLS + k
        pltpu.make_async_copy(dfqs[k % 2], dfq_hbm.at[j], sem_out).wait()

    pltpu.sync_copy(accq, outq.at[wid])
    pltpu.sync_copy(accp, outp.at[wid])


def _finish_body(q_ref, p_ref, dfq_ref, out_ref):
    qtf = jnp.sum(q_ref[...], axis=0).astype(jnp.float32)
    ptf = jnp.sum(p_ref[...], axis=0).astype(jnp.float32)
    w = (qtf / (K3 + qtf)) * (K1 * ptf / (ptf + _DENOM))
    df = dfq_ref[...]
    idf = jnp.log2((N_DOCS - df + 0.5) / (df + 0.5))
    score = jnp.sum(idf * w[None, :], axis=1)
    out_ref[...] = (1.0 / (1.0 + jnp.exp(-score)))[None, :]


def _finish(q_parts, p_parts, dfq):
    return pl.pallas_call(
        _finish_body,
        out_shape=jax.ShapeDtypeStruct((1, LPAD), jnp.float32),
    )(q_parts, p_parts, dfq)


@jax.jit
def kernel(ids, masks, DF_table):
    del masks
    qT = jnp.transpose(ids[0])
    pT = jnp.transpose(ids[1])
    q_parts, p_parts, dfq = _count_kernel(qT, pT, DF_table)
    return _finish(q_parts, p_parts, dfq)[0, :L_SEQ]


# speedup vs baseline: 86.5969x; 1.0150x over previous
import functools

import jax
import jax.numpy as jnp
from jax import lax
from jax.experimental import pallas as pl
from jax.experimental.pallas import tpu as pltpu
from jax.experimental.pallas import tpu_sc as plsc

K1 = 1.2
K3 = 8.0
B_PARAM = 0.75
N_DOCS = 8800000.0
LAVE = 250.0
VOCAB = 100000
BATCH = 1024
L_SEQ = 200

NC = 2
NS = 16
NW = NC * NS
LANES = 16
COLS = 7
LPAD = NW * COLS
HIST_PAD = 100352
VREGS_B = BATCH // LANES
UNROLL = 16
IDX_CHUNK = 128
PHI = 65536

_DENOM = K1 * (1.0 - B_PARAM + B_PARAM * L_SEQ / LAVE)

_mesh = plsc.VectorSubcoreMesh(core_axis_name="c", subcore_axis_name="s")
_sc_params = pltpu.CompilerParams(
    needs_layout_passes=False,
    disable_bounds_checks=True,
    skip_device_barrier=True,
)


@functools.partial(
    pl.kernel,
    mesh=_mesh,
    out_type=(
        jax.ShapeDtypeStruct((NW, BATCH), jnp.int32),
        jax.ShapeDtypeStruct((LPAD, BATCH), jnp.float32),
    ),
    scratch_types=[
        pltpu.VMEM((HIST_PAD,), jnp.int32),
        pltpu.VMEM((BATCH,), jnp.int32),
        pltpu.VMEM((BATCH,), jnp.int32),
        pltpu.VMEM((BATCH,), jnp.int32),
        pltpu.VMEM((BATCH,), jnp.int32),
        pltpu.VMEM((BATCH,), jnp.int32),
        pltpu.VMEM((BATCH,), jnp.float32),
        pltpu.VMEM((BATCH,), jnp.float32),
        pltpu.SemaphoreType.DMA,
        pltpu.SemaphoreType.DMA,
        pltpu.SemaphoreType.DMA,
    ],
    compiler_params=_sc_params,
)
def _count_kernel(qT, pT, df_hbm, outc, dfq_hbm,
                  hist, qcol_a, qcol_b, pcol_a, pcol_b, acc,
                  dfq_a, dfq_b, sem_in, sem_g, sem_out):
    qcols = (qcol_a, qcol_b)
    pcols = (pcol_a, pcol_b)
    dfqs = (dfq_a, dfq_b)
    wid = lax.axis_index("s") * NC + lax.axis_index("c")
    ones = jnp.full((LANES,), 1, jnp.int32)
    neg_ones = jnp.full((LANES,), -1, jnp.int32)
    p_ones = jnp.full((LANES,), PHI, jnp.int32)
    neg_p_ones = jnp.full((LANES,), -PHI, jnp.int32)
    zeros16 = jnp.zeros((LANES,), jnp.int32)

    def fire_loads(k):
        j = wid * COLS + k

        @pl.when(j < L_SEQ)
        def _():
            pltpu.async_copy(qT.at[j], qcols[k % 2], sem_in)
            pltpu.async_copy(pT.at[j], pcols[k % 2], sem_in)

    def wait_loads(k):
        j = wid * COLS + k

        @pl.when(j < L_SEQ)
        def _():
            pltpu.make_async_copy(qT.at[j], qcols[k % 2], sem_in).wait()
            pltpu.make_async_copy(pT.at[j], pcols[k % 2], sem_in).wait()

    fire_loads(0)

    def zero_hist(i, carry):
        for u in range(16):
            hist[pl.ds(i * 256 + u * 16, LANES)] = zeros16
        return carry

    lax.fori_loop(0, HIST_PAD // 256, zero_hist, 0)

    def zero_acc(i, carry):
        acc[pl.ds(i * LANES, LANES)] = zeros16
        dfq_a[pl.ds(i * LANES, LANES)] = jnp.zeros((LANES,), jnp.float32)
        dfq_b[pl.ds(i * LANES, LANES)] = jnp.zeros((LANES,), jnp.float32)
        return carry

    lax.fori_loop(0, VREGS_B, zero_acc, 0)

    for k in range(COLS):
        j = wid * COLS + k
        qcol = qcols[k % 2]
        pcol = pcols[k % 2]
        dfq = dfqs[k % 2]

        wait_loads(k)
        if k + 1 < COLS:
            fire_loads(k + 1)
        if k >= 2:
            jm2 = wid * COLS + (k - 2)
            pltpu.make_async_copy(dfq, dfq_hbm.at[jm2], sem_out).wait()

        @pl.when(j < L_SEQ)
        def _():
            for c in range(BATCH // IDX_CHUNK):
                sl = pl.ds(c * IDX_CHUNK, IDX_CHUNK)
                pltpu.async_copy(df_hbm.at[qcol.at[sl]], dfq.at[sl], sem_g)

            def scatter_q(delta):
                def body(i, c):
                    for u in range(UNROLL):
                        idx = qcol[pl.ds(i * (UNROLL * LANES) + u * LANES,
                                         LANES)]
                        plsc.addupdate_scatter(hist, [idx], delta)
                    return c

                lax.fori_loop(0, VREGS_B // UNROLL, body, 0)

            def scatter_p(delta):
                def body(i, c):
                    for u in range(UNROLL):
                        idx = pcol[pl.ds(i * (UNROLL * LANES) + u * LANES,
                                         LANES)]
                        plsc.addupdate_scatter(hist, [idx], delta)
                    return c

                lax.fori_loop(0, VREGS_B // UNROLL, body, 0)

            def gather_both(i, c):
                for u in range(UNROLL):
                    sl = pl.ds(i * (UNROLL * LANES) + u * LANES, LANES)
                    idx = qcol[sl]
                    acc[sl] = acc[sl] + plsc.load_gather(hist, [idx])
                return c

            scatter_q(ones)
            scatter_p(p_ones)
            lax.fori_loop(0, VREGS_B // UNROLL, gather_both, 0)
            if k + 1 < COLS:
                scatter_q(neg_ones)
                scatter_p(neg_p_ones)

            for c in range(BATCH // IDX_CHUNK):
                sl = pl.ds(c * IDX_CHUNK, IDX_CHUNK)
                pltpu.make_async_copy(
                    df_hbm.at[qcol.at[sl]], dfq.at[sl], sem_g
                ).wait()

        pltpu.async_copy(dfq, dfq_hbm.at[j], sem_out)

    for k in (COLS - 2, COLS - 1):
        j = wid * COLS + k
        pltpu.make_async_copy(dfqs[k % 2], dfq_hbm.at[j], sem_out).wait()

    pltpu.sync_copy(acc, outc.at[wid])


def _finish_body(c_ref, dfq_ref, out_ref):
    packed = c_ref[...]
    qtf = jnp.sum(packed & (PHI - 1), axis=0).astype(jnp.float32)
    ptf = jnp.sum(packed >> 16, axis=0).astype(jnp.float32)
    w = (qtf / (K3 + qtf)) * (K1 * ptf / (ptf + _DENOM))
    df = dfq_ref[...]
    idf = jnp.log2((N_DOCS - df + 0.5) / (df + 0.5))
    score = jnp.sum(idf * w[None, :], axis=1)
    out_ref[...] = (1.0 / (1.0 + jnp.exp(-score)))[None, :]


def _finish(c_parts, dfq):
    return pl.pallas_call(
        _finish_body,
        out_shape=jax.ShapeDtypeStruct((1, LPAD), jnp.float32),
    )(c_parts, dfq)


@jax.jit
def kernel(ids, masks, DF_table):
    del masks
    qT = jnp.transpose(ids[0])
    pT = jnp.transpose(ids[1])
    c_parts, dfq = _count_kernel(qT, pT, DF_table)
    return _finish(c_parts, dfq)[0, :L_SEQ]


# speedup vs baseline: 93.6895x; 1.0819x over previous
import functools

import jax
import jax.numpy as jnp
from jax import lax
from jax.experimental import pallas as pl
from jax.experimental.pallas import tpu as pltpu
from jax.experimental.pallas import tpu_sc as plsc

K1 = 1.2
K3 = 8.0
B_PARAM = 0.75
N_DOCS = 8800000.0
LAVE = 250.0
VOCAB = 100000
BATCH = 1024
L_SEQ = 200

NC = 2
NS = 16
NW = NC * NS
LANES = 16
COLS = 7
LPAD = NW * COLS
HIST_PAD = 100352
VREGS_B = BATCH // LANES
UNROLL = 16
IDX_CHUNK = 128
PHI = 65536

_DENOM = K1 * (1.0 - B_PARAM + B_PARAM * L_SEQ / LAVE)

_mesh = plsc.VectorSubcoreMesh(core_axis_name="c", subcore_axis_name="s")
_sc_params = pltpu.CompilerParams(
    needs_layout_passes=False,
    disable_bounds_checks=True,
    skip_device_barrier=True,
)


@functools.partial(
    pl.kernel,
    mesh=_mesh,
    out_type=(
        jax.ShapeDtypeStruct((NW, BATCH), jnp.int32),
        jax.ShapeDtypeStruct((LPAD, BATCH), jnp.float32),
    ),
    scratch_types=[
        pltpu.VMEM((HIST_PAD,), jnp.int32),
        pltpu.VMEM((BATCH,), jnp.int32),
        pltpu.VMEM((BATCH,), jnp.int32),
        pltpu.VMEM((BATCH,), jnp.int32),
        pltpu.VMEM((BATCH,), jnp.int32),
        pltpu.VMEM((BATCH,), jnp.int32),
        pltpu.VMEM((BATCH,), jnp.float32),
        pltpu.VMEM((BATCH,), jnp.float32),
        pltpu.SemaphoreType.DMA,
        pltpu.SemaphoreType.DMA,
        pltpu.SemaphoreType.DMA,
    ],
    compiler_params=_sc_params,
)
def _count_kernel(qT, pT, df_hbm, outc, dfq_hbm,
                  hist, qcol_a, qcol_b, pcol_a, pcol_b, acc,
                  dfq_a, dfq_b, sem_in, sem_g, sem_out):
    qcols = (qcol_a, qcol_b)
    pcols = (pcol_a, pcol_b)
    dfqs = (dfq_a, dfq_b)
    wid = lax.axis_index("s") * NC + lax.axis_index("c")
    ones = jnp.full((LANES,), 1, jnp.int32)
    neg_ones = jnp.full((LANES,), -1, jnp.int32)
    p_ones = jnp.full((LANES,), PHI, jnp.int32)
    neg_p_ones = jnp.full((LANES,), -PHI, jnp.int32)
    zeros16 = jnp.zeros((LANES,), jnp.int32)

    def fire_loads(k):
        j = wid * COLS + k

        @pl.when(j < L_SEQ)
        def _():
            pltpu.async_copy(qT.at[j], qcols[k % 2], sem_in)
            pltpu.async_copy(pT.at[j], pcols[k % 2], sem_in)

    def wait_loads(k):
        j = wid * COLS + k

        @pl.when(j < L_SEQ)
        def _():
            pltpu.make_async_copy(qT.at[j], qcols[k % 2], sem_in).wait()
            pltpu.make_async_copy(pT.at[j], pcols[k % 2], sem_in).wait()

    fire_loads(0)

    @plsc.parallel_loop(0, HIST_PAD // LANES, unroll=16)
    def zero_hist(i):
        hist[pl.ds(i * LANES, LANES)] = zeros16

    @plsc.parallel_loop(0, VREGS_B, unroll=8)
    def zero_acc(i):
        acc[pl.ds(i * LANES, LANES)] = zeros16
        dfq_a[pl.ds(i * LANES, LANES)] = jnp.zeros((LANES,), jnp.float32)
        dfq_b[pl.ds(i * LANES, LANES)] = jnp.zeros((LANES,), jnp.float32)

    for k in range(COLS):
        j = wid * COLS + k
        qcol = qcols[k % 2]
        pcol = pcols[k % 2]
        dfq = dfqs[k % 2]

        wait_loads(k)
        if k + 1 < COLS:
            fire_loads(k + 1)
        if k >= 2:
            jm2 = wid * COLS + (k - 2)
            pltpu.make_async_copy(dfq, dfq_hbm.at[jm2], sem_out).wait()

        @pl.when(j < L_SEQ)
        def _():
            for c in range(BATCH // IDX_CHUNK):
                sl = pl.ds(c * IDX_CHUNK, IDX_CHUNK)
                pltpu.async_copy(df_hbm.at[qcol.at[sl]], dfq.at[sl], sem_g)

            def scatter_q(delta):
                @plsc.parallel_loop(0, VREGS_B, unroll=UNROLL)
                def body(i):
                    idx = qcol[pl.ds(i * LANES, LANES)]
                    plsc.addupdate_scatter(hist, [idx], delta)

            def scatter_p(delta):
                @plsc.parallel_loop(0, VREGS_B, unroll=UNROLL)
                def body(i):
                    idx = pcol[pl.ds(i * LANES, LANES)]
                    plsc.addupdate_scatter(hist, [idx], delta)

            scatter_q(ones)
            scatter_p(p_ones)

            @plsc.parallel_loop(0, VREGS_B, unroll=UNROLL)
            def gather_both(i):
                sl = pl.ds(i * LANES, LANES)
                idx = qcol[sl]
                acc[sl] = acc[sl] + plsc.load_gather(hist, [idx])

            if k + 1 < COLS:
                scatter_q(neg_ones)
                scatter_p(neg_p_ones)

            for c in range(BATCH // IDX_CHUNK):
                sl = pl.ds(c * IDX_CHUNK, IDX_CHUNK)
                pltpu.make_async_copy(
                    df_hbm.at[qcol.at[sl]], dfq.at[sl], sem_g
                ).wait()

        pltpu.async_copy(dfq, dfq_hbm.at[j], sem_out)

    for k in (COLS - 2, COLS - 1):
        j = wid * COLS + k
        pltpu.make_async_copy(dfqs[k % 2], dfq_hbm.at[j], sem_out).wait()

    pltpu.sync_copy(acc, outc.at[wid])


def _finish_body(c_ref, dfq_ref, out_ref):
    packed = c_ref[...]
    qtf = jnp.sum(packed & (PHI - 1), axis=0).astype(jnp.float32)
    ptf = jnp.sum(packed >> 16, axis=0).astype(jnp.float32)
    w = (qtf / (K3 + qtf)) * (K1 * ptf / (ptf + _DENOM))
    df = dfq_ref[...]
    idf = jnp.log2((N_DOCS - df + 0.5) / (df + 0.5))
    score = jnp.sum(idf * w[None, :], axis=1)
    out_ref[...] = (1.0 / (1.0 + jnp.exp(-score)))[None, :]


def _finish(c_parts, dfq):
    return pl.pallas_call(
        _finish_body,
        out_shape=jax.ShapeDtypeStruct((1, LPAD), jnp.float32),
    )(c_parts, dfq)


@jax.jit
def kernel(ids, masks, DF_table):
    del masks
    qT = jnp.transpose(ids[0])
    pT = jnp.transpose(ids[1])
    c_parts, dfq = _count_kernel(qT, pT, DF_table)
    return _finish(c_parts, dfq)[0, :L_SEQ]


# speedup vs baseline: 98.2113x; 1.0483x over previous
import functools

import jax
import jax.numpy as jnp
from jax import lax
from jax.experimental import pallas as pl
from jax.experimental.pallas import tpu as pltpu
from jax.experimental.pallas import tpu_sc as plsc

K1 = 1.2
K3 = 8.0
B_PARAM = 0.75
N_DOCS = 8800000.0
LAVE = 250.0
VOCAB = 100000
BATCH = 1024
L_SEQ = 200

NC = 2
NS = 16
NW = NC * NS
LANES = 16
COLS = 7
LPAD = NW * COLS
HIST_PAD = 100352
VREGS_B = BATCH // LANES
UNROLL = 16
IDX_CHUNK = 128
PHI = 65536

_DENOM = K1 * (1.0 - B_PARAM + B_PARAM * L_SEQ / LAVE)

_mesh = plsc.VectorSubcoreMesh(core_axis_name="c", subcore_axis_name="s")
_sc_params = pltpu.CompilerParams(
    needs_layout_passes=False,
    disable_bounds_checks=True,
    skip_device_barrier=True,
)


@functools.partial(
    pl.kernel,
    mesh=_mesh,
    out_type=(
        jax.ShapeDtypeStruct((NW, BATCH), jnp.int32),
        jax.ShapeDtypeStruct((LPAD, BATCH), jnp.float32),
    ),
    scratch_types=[
        pltpu.VMEM((HIST_PAD,), jnp.int32),
        pltpu.VMEM((2 * BATCH,), jnp.int32),
        pltpu.VMEM((2 * BATCH,), jnp.int32),
        pltpu.VMEM((BATCH,), jnp.int32),
        pltpu.VMEM((BATCH,), jnp.float32),
        pltpu.VMEM((BATCH,), jnp.float32),
        pltpu.SemaphoreType.DMA,
        pltpu.SemaphoreType.DMA,
        pltpu.SemaphoreType.DMA,
    ],
    compiler_params=_sc_params,
)
def _count_kernel(qT, pT, df_hbm, outc, dfq_hbm,
                  hist, qp_a, qp_b, acc,
                  dfq_a, dfq_b, sem_in, sem_g, sem_out):
    qps = (qp_a, qp_b)
    dfqs = (dfq_a, dfq_b)
    wid = lax.axis_index("s") * NC + lax.axis_index("c")
    ones = jnp.full((LANES,), 1, jnp.int32)
    neg_ones = jnp.full((LANES,), -1, jnp.int32)
    p_ones = jnp.full((LANES,), PHI, jnp.int32)
    neg_p_ones = jnp.full((LANES,), -PHI, jnp.int32)
    zeros16 = jnp.zeros((LANES,), jnp.int32)

    def fire_loads(k):
        j = wid * COLS + k

        @pl.when(j < L_SEQ)
        def _():
            qp = qps[k % 2]
            pltpu.async_copy(qT.at[j], qp.at[pl.ds(0, BATCH)], sem_in)
            pltpu.async_copy(pT.at[j], qp.at[pl.ds(BATCH, BATCH)], sem_in)

    def wait_loads(k):
        j = wid * COLS + k

        @pl.when(j < L_SEQ)
        def _():
            qp = qps[k % 2]
            pltpu.make_async_copy(
                qT.at[j], qp.at[pl.ds(0, BATCH)], sem_in).wait()
            pltpu.make_async_copy(
                pT.at[j], qp.at[pl.ds(BATCH, BATCH)], sem_in).wait()

    fire_loads(0)

    @plsc.parallel_loop(0, HIST_PAD // LANES, unroll=16)
    def zero_hist(i):
        hist[pl.ds(i * LANES, LANES)] = zeros16

    @plsc.parallel_loop(0, VREGS_B, unroll=8)
    def zero_acc(i):
        acc[pl.ds(i * LANES, LANES)] = zeros16
        dfq_a[pl.ds(i * LANES, LANES)] = jnp.zeros((LANES,), jnp.float32)
        dfq_b[pl.ds(i * LANES, LANES)] = jnp.zeros((LANES,), jnp.float32)

    for k in range(COLS):
        j = wid * COLS + k
        qp = qps[k % 2]
        dfq = dfqs[k % 2]

        wait_loads(k)
        if k + 1 < COLS:
            fire_loads(k + 1)
        if k >= 2:
            jm2 = wid * COLS + (k - 2)
            pltpu.make_async_copy(dfq, dfq_hbm.at[jm2], sem_out).wait()

        @pl.when(j < L_SEQ)
        def _():
            for c in range(BATCH // IDX_CHUNK):
                sl = pl.ds(c * IDX_CHUNK, IDX_CHUNK)
                pltpu.async_copy(df_hbm.at[qp.at[sl]], dfq.at[sl], sem_g)

            def scatter_qp(sign):
                @plsc.parallel_loop(0, 2 * VREGS_B, unroll=UNROLL)
                def body(i):
                    idx = qp[pl.ds(i * LANES, LANES)]
                    is_q = jnp.broadcast_to(i, (LANES,)) < VREGS_B
                    delta = jnp.where(is_q, sign, sign * PHI)
                    plsc.addupdate_scatter(hist, [idx], delta)

            scatter_qp(1)

            @plsc.parallel_loop(0, VREGS_B, unroll=UNROLL)
            def gather_both(i):
                sl = pl.ds(i * LANES, LANES)
                idx = qp[sl]
                acc[sl] = acc[sl] + plsc.load_gather(hist, [idx])

            if k + 1 < COLS:
                scatter_qp(-1)

            for c in range(BATCH // IDX_CHUNK):
                sl = pl.ds(c * IDX_CHUNK, IDX_CHUNK)
                pltpu.make_async_copy(
                    df_hbm.at[qp.at[sl]], dfq.at[sl], sem_g
                ).wait()

        pltpu.async_copy(dfq, dfq_hbm.at[j], sem_out)

    for k in (COLS - 2, COLS - 1):
        j = wid * COLS + k
        pltpu.make_async_copy(dfqs[k % 2], dfq_hbm.at[j], sem_out).wait()

    pltpu.sync_copy(acc, outc.at[wid])


def _finish_body(c_ref, dfq_ref, out_ref):
    packed = c_ref[...]
    qtf = jnp.sum(packed & (PHI - 1), axis=0).astype(jnp.float32)
    ptf = jnp.sum(packed >> 16, axis=0).astype(jnp.float32)
    w = (qtf / (K3 + qtf)) * (K1 * ptf / (ptf + _DENOM))
    df = dfq_ref[...]
    idf = jnp.log2((N_DOCS - df + 0.5) / (df + 0.5))
    score = jnp.sum(idf * w[None, :], axis=1)
    out_ref[...] = (1.0 / (1.0 + jnp.exp(-score)))[None, :]


def _finish(c_parts, dfq):
    return pl.pallas_call(
        _finish_body,
        out_shape=jax.ShapeDtypeStruct((1, LPAD), jnp.float32),
    )(c_parts, dfq)


@jax.jit
def kernel(ids, masks, DF_table):
    del masks
    qT = jnp.transpose(ids[0])
    pT = jnp.transpose(ids[1])
    c_parts, dfq = _count_kernel(qT, pT, DF_table)
    return _finish(c_parts, dfq)[0, :L_SEQ]
